# Initial kernel scaffold; baseline (speedup 1.0000x reference)
import jax
import jax.numpy as jnp
from jax.experimental import pallas as pl


def kernel(events_x, events_y, calib_center, precompute_grid, pairwise_dists_mask, pairwise_dists, correction):
    raise NotImplementedError("write your pallas kernel here")


# speedup vs baseline: 591.7205x; 591.7205x over previous
import functools

import jax
import jax.numpy as jnp
from jax import lax
from jax.experimental import pallas as pl
from jax.experimental.pallas import tpu as pltpu
from jax.experimental.pallas import tpu_sc as plsc

N = 2048
E = 4096
G = 101
TAB = 10208
NC = 2
NS = 16
NW = NC * NS
L = 16
DPW = N // NW
GROUPS = DPW // L


def _sc_event_reduce(exf, eyf, cx, cy, tab0, tab1):
    mesh = plsc.VectorSubcoreMesh(core_axis_name="c", subcore_axis_name="s")
    out_type = (
        jax.ShapeDtypeStruct((N,), jnp.float32),
        jax.ShapeDtypeStruct((N,), jnp.float32),
        jax.ShapeDtypeStruct((N,), jnp.float32),
    )
    scratch = [
        pltpu.VMEM((E,), jnp.float32),
        pltpu.VMEM((E,), jnp.float32),
        pltpu.VMEM((TAB,), jnp.float32),
        pltpu.VMEM((TAB,), jnp.float32),
        pltpu.VMEM((TAB,), jnp.float32),
        pltpu.VMEM((DPW,), jnp.float32),
        pltpu.VMEM((DPW,), jnp.float32),
        pltpu.VMEM((DPW,), jnp.float32),
        pltpu.VMEM((DPW,), jnp.float32),
        pltpu.VMEM((DPW,), jnp.float32),
    ]

    @functools.partial(pl.kernel, out_type=out_type, mesh=mesh,
                       scratch_types=scratch,
                       compiler_params=pltpu.CompilerParams(
                           needs_layout_passes=False,
                           use_tc_tiling_on_sc=False))
    def k(ex_h, ey_h, cx_h, cy_h, t0_h, t1_h, s0_h, s1_h, ct_h,
          ex_v, ey_v, t0_v, t1_v, tc_v, cx_v, cy_v, s0_v, s1_v, ct_v):
        wid = lax.axis_index("s") * NC + lax.axis_index("c")
        base = wid * DPW
        pltpu.sync_copy(ex_h, ex_v)
        pltpu.sync_copy(ey_h, ey_v)
        pltpu.sync_copy(t0_h, t0_v)
        pltpu.sync_copy(t1_h, t1_v)
        pltpu.sync_copy(cx_h.at[pl.ds(base, DPW)], cx_v)
        pltpu.sync_copy(cy_h.at[pl.ds(base, DPW)], cy_v)

        def build(kk, _):
            sl = pl.ds(kk * L, L)
            t0 = t0_v[sl]
            t1 = t1_v[sl]
            one = jnp.ones((L,), jnp.float32)
            zro = jnp.zeros((L,), jnp.float32)
            cnt = (jnp.where(t0 != 0.0, one, zro)
                   + jnp.where(t1 != 0.0, one, zro))
            tc_v[sl] = cnt
            return 0
        lax.fori_loop(0, TAB // L, build, 0, unroll=2)

        zero = jnp.zeros((L,), jnp.float32)
        for g in range(GROUPS):
            gsl = pl.ds(g * L, L)
            cxv = cx_v[gsl]
            cyv = cy_v[gsl]

            def body(j, carry, cxv=cxv, cyv=cyv):
                s0, s1, sc = carry
                jb = jnp.full((L,), j, jnp.int32)
                exb = plsc.load_gather(ex_v, [jb])
                eyb = plsc.load_gather(ey_v, [jb])
                dx = (exb - cxv).astype(jnp.int32)
                dy = (eyb - cyv).astype(jnp.int32)
                u = jnp.clip(dx, -50, 50)
                v = jnp.clip(dy, -50, 50)
                flat = u * G + v + (50 * G + 50)
                g0 = plsc.load_gather(t0_v, [flat])
                g1 = plsc.load_gather(t1_v, [flat])
                gc = plsc.load_gather(tc_v, [flat])
                return s0 + g0, s1 + g1, sc + gc

            s0, s1, sc = lax.fori_loop(0, E, body, (zero, zero, zero),
                                       unroll=4)
            s0_v[gsl] = s0
            s1_v[gsl] = s1
            ct_v[gsl] = sc

        pltpu.sync_copy(s0_v, s0_h.at[pl.ds(base, DPW)])
        pltpu.sync_copy(s1_v, s1_h.at[pl.ds(base, DPW)])
        pltpu.sync_copy(ct_v, ct_h.at[pl.ds(base, DPW)])

    return k(exf, eyf, cx, cy, tab0, tab1)


def _tc_combine(cx, cy, mask, dists, corr, s0, s1, cnt):
    BR = 256
    grid = (N // BR,)
    full = pl.BlockSpec((N,), lambda i: (0,))
    big = pl.BlockSpec((BR, N), lambda i: (i, 0))
    rows = pl.BlockSpec((BR,), lambda i: (i,))

    def body(cx_ref, cy_ref, mask_ref, dists_ref, corr_ref,
             s0_ref, s1_ref, ct_ref, nx_ref, ny_ref):
        i = pl.program_id(0)
        r = pl.ds(i * BR, BR)
        cxr = cx_ref[r]
        cyr = cy_ref[r]
        dxc = cx_ref[...][None, :] - cxr[:, None]
        dyc = cy_ref[...][None, :] - cyr[:, None]
        m = mask_ref[...]
        d = dists_ref[...]
        sdx = dxc * m
        sdy = dyc * m
        radi = sdx * sdx + sdy * sdy - d * d
        rsx = jnp.sum(4.0 * dxc * radi, axis=1)
        rsy = jnp.sum(4.0 * dyc * radi, axis=1)
        corr_r = corr_ref[r]
        cdu = corr_r * rsx
        cdv = corr_r * rsy
        dec = (ct_ref[r] >= 10.0).astype(jnp.float32)
        scale = 200 * 1.5e-05
        nx_ref[...] = cxr - scale * dec * (
            jnp.clip(s0_ref[r], -400.0, 400.0) - 2.5e-07 * cdu)
        ny_ref[...] = cyr - scale * dec * (
            jnp.clip(s1_ref[r], -400.0, 400.0) - 2.5e-07 * cdv)

    return pl.pallas_call(
        body,
        grid=grid,
        in_specs=[full, full, big, big, full, full, full, full],
        out_specs=[rows, rows],
        out_shape=[
            jax.ShapeDtypeStruct((N,), jnp.float32),
            jax.ShapeDtypeStruct((N,), jnp.float32),
        ],
    )(cx, cy, mask, dists, corr, s0, s1, cnt)


def kernel(events_x, events_y, calib_center, precompute_grid,
           pairwise_dists_mask, pairwise_dists, correction):
    exf = events_x.astype(jnp.float32)
    eyf = events_y.astype(jnp.float32)
    cx = calib_center[:, 1]
    cy = calib_center[:, 0]
    flat = precompute_grid.reshape(G * G, 2)
    pad = TAB - G * G
    tab0 = jnp.pad(flat[:, 0], (0, pad))
    tab1 = jnp.pad(flat[:, 1], (0, pad))
    s0, s1, cnt = _sc_event_reduce(exf, eyf, cx, cy, tab0, tab1)
    nx, ny = _tc_combine(cx, cy, pairwise_dists_mask, pairwise_dists,
                         correction, s0, s1, cnt)
    return jnp.stack([ny, nx], axis=1)
